# baseline (speedup 1.0000x reference)
import jax
import jax.numpy as jnp
from jax.experimental import pallas as pl
from jax.experimental.pallas import tpu as pltpu

_K = 3
_ITERS = 5
_ROWS = 2048
_COLS = 128


def _kmeans_body(c_ref, x_ref, y_ref, z_ref, o0_ref, o1_ref, o2_ref):
    x = x_ref[...]
    y = y_ref[...]
    z = z_ref[...]
    nn = jnp.float32(_ROWS * _COLS)
    sx_t = jnp.sum(x)
    sy_t = jnp.sum(y)
    sz_t = jnp.sum(z)

    def masks_from_centers(c):
        c0x, c0y, c0z, c1x, c1y, c1z, c2x, c2y, c2z = c
        d0 = (x - c0x) ** 2 + (y - c0y) ** 2 + (z - c0z) ** 2
        d1 = (x - c1x) ** 2 + (y - c1y) ** 2 + (z - c1z) ** 2
        d2 = (x - c2x) ** 2 + (y - c2y) ** 2 + (z - c2z) ** 2
        lt1 = d1 < d0
        sel2 = d2 < jnp.minimum(d0, d1)
        sel1 = jnp.logical_and(lt1, jnp.logical_not(sel2))
        sel0 = jnp.logical_and(jnp.logical_not(lt1), jnp.logical_not(sel2))
        return sel0, sel1, sel2

    def body(_, c):
        sel0, sel1, _ = masks_from_centers(c)
        f0 = sel0.astype(jnp.float32)
        f1 = sel1.astype(jnp.float32)
        n0 = jnp.sum(f0)
        n1 = jnp.sum(f1)
        n2 = nn - n0 - n1
        sx0 = jnp.sum(x * f0)
        sy0 = jnp.sum(y * f0)
        sz0 = jnp.sum(z * f0)
        sx1 = jnp.sum(x * f1)
        sy1 = jnp.sum(y * f1)
        sz1 = jnp.sum(z * f1)
        return (sx0 / n0, sy0 / n0, sz0 / n0,
                sx1 / n1, sy1 / n1, sz1 / n1,
                (sx_t - sx0 - sx1) / n2,
                (sy_t - sy0 - sy1) / n2,
                (sz_t - sz0 - sz1) / n2)

    c0 = tuple(c_ref[i, j] for i in range(_K) for j in range(3))
    c_fin = jax.lax.fori_loop(0, _ITERS - 1, body, c0)
    sel0, _, _ = masks_from_centers(c_fin)
    zeros = jnp.zeros_like(x)
    o0_ref[...] = jnp.where(sel0, 0.0, zeros)
    o1_ref[...] = jnp.where(sel0, 0.0, zeros)
    o2_ref[...] = jnp.where(sel0, 0.0, zeros)


def kernel(data, img_shape):
    del img_shape
    data = data.reshape((-1, 3))
    n = data.shape[0]
    init_idx = jax.random.randint(jax.random.key(42), (3,), 0, n)
    centers = jnp.take(data, init_idx, axis=0)
    planes = data.T.reshape(3, _ROWS, _COLS)
    x, y, z = planes[0], planes[1], planes[2]

    out_shape = jax.ShapeDtypeStruct((_ROWS, _COLS), jnp.float32)
    o0, o1, o2 = pl.pallas_call(
        _kmeans_body,
        in_specs=[
            pl.BlockSpec(memory_space=pltpu.SMEM),
            pl.BlockSpec(memory_space=pltpu.VMEM),
            pl.BlockSpec(memory_space=pltpu.VMEM),
            pl.BlockSpec(memory_space=pltpu.VMEM),
        ],
        out_specs=[
            pl.BlockSpec(memory_space=pltpu.VMEM),
            pl.BlockSpec(memory_space=pltpu.VMEM),
            pl.BlockSpec(memory_space=pltpu.VMEM),
        ],
        out_shape=[out_shape, out_shape, out_shape],
    )(centers, x, y, z)

    out = jnp.stack([o0.reshape(n), o1.reshape(n), o2.reshape(n)], axis=-1)
    return out.reshape(n, 1, 3)
